# baseline (speedup 1.0000x reference)
import jax
import jax.numpy as jnp
from jax.experimental import pallas as pl

NEG_INF = float("-inf")
BLK = 32


def _nms_body(x_ref, o_ref):
    def one_plane(p, carry):
        x = x_ref[p]
        left = jnp.concatenate([jnp.full_like(x[:, :1], NEG_INF), x[:, :-1]], axis=1)
        right = jnp.concatenate([x[:, 1:], jnp.full_like(x[:, :1], NEG_INF)], axis=1)
        rowmax = jnp.maximum(jnp.maximum(left, x), right)
        up = jnp.concatenate([jnp.full_like(rowmax[:1, :], NEG_INF), rowmax[:-1, :]], axis=0)
        down = jnp.concatenate([rowmax[1:, :], jnp.full_like(rowmax[:1, :], NEG_INF)], axis=0)
        hmax = jnp.maximum(jnp.maximum(up, rowmax), down)
        o_ref[p] = jnp.where(hmax == x, x, 0.0)
        return carry

    jax.lax.fori_loop(0, BLK, one_plane, 0, unroll=False)


def kernel(points):
    n, c, h, w = points.shape
    x = points.reshape(n * c, h, w)
    out = pl.pallas_call(
        _nms_body,
        grid=((n * c) // BLK,),
        in_specs=[pl.BlockSpec((BLK, h, w), lambda i: (i, 0, 0))],
        out_specs=pl.BlockSpec((BLK, h, w), lambda i: (i, 0, 0)),
        out_shape=jax.ShapeDtypeStruct((n * c, h, w), points.dtype),
    )(x)
    return out.reshape(n, c, h, w)


# speedup vs baseline: 1.0525x; 1.0525x over previous
import jax
import jax.numpy as jnp
from jax.experimental import pallas as pl
from jax.experimental.pallas import tpu as pltpu

NEG_INF = float("-inf")
BLK = 32


def _nms_body(x_ref, o_ref):
    x = x_ref[...]
    shape = x.shape
    col = jax.lax.broadcasted_iota(jnp.int32, shape, 2)
    row = jax.lax.broadcasted_iota(jnp.int32, shape, 1)
    w = shape[2]
    h = shape[1]
    left = jnp.where(col == 0, NEG_INF, pltpu.roll(x, 1, 2))
    right = jnp.where(col == w - 1, NEG_INF, pltpu.roll(x, w - 1, 2))
    rowmax = jnp.maximum(jnp.maximum(left, x), right)
    up = jnp.where(row == 0, NEG_INF, pltpu.roll(rowmax, 1, 1))
    down = jnp.where(row == h - 1, NEG_INF, pltpu.roll(rowmax, h - 1, 1))
    hmax = jnp.maximum(jnp.maximum(up, rowmax), down)
    o_ref[...] = jnp.where(hmax == x, x, 0.0)


def kernel(points):
    n, c, h, w = points.shape
    x = points.reshape(n * c, h, w)
    out = pl.pallas_call(
        _nms_body,
        grid=((n * c) // BLK,),
        in_specs=[pl.BlockSpec((BLK, h, w), lambda i: (i, 0, 0))],
        out_specs=pl.BlockSpec((BLK, h, w), lambda i: (i, 0, 0)),
        out_shape=jax.ShapeDtypeStruct((n * c, h, w), points.dtype),
    )(x)
    return out.reshape(n, c, h, w)
